# baseline (speedup 1.0000x reference)
import functools

import jax
import jax.numpy as jnp
from jax import lax
from jax.experimental import pallas as pl
from jax.experimental.pallas import tpu as pltpu
from jax.experimental.pallas import tpu_sc as plsc

N = 10000
D = 128
NP = 10240
NC = 2
NS = 16
NW = NC * NS
C = 64
C2 = 128
ROWS_PER_TILE = NP // NS


def _dual_matmul_body(x_ref, wa_ref, wb_ref, a_ref, b_ref):
    xv = x_ref[...]
    dn = (((1,), (1,)), ((), ()))
    a_ref[...] = lax.dot_general(xv, wa_ref[...], dn, preferred_element_type=jnp.float32)
    b_ref[...] = lax.dot_general(xv, wb_ref[...], dn, preferred_element_type=jnp.float32)


def _tc_dual_matmul(x, wa, wb, br=2048):
    n = x.shape[0]
    return pl.pallas_call(
        _dual_matmul_body,
        grid=(n // br,),
        in_specs=[
            pl.BlockSpec((br, D), lambda i: (i, 0)),
            pl.BlockSpec((D, D), lambda i: (0, 0)),
            pl.BlockSpec((D, D), lambda i: (0, 0)),
        ],
        out_specs=[
            pl.BlockSpec((br, D), lambda i: (i, 0)),
            pl.BlockSpec((br, D), lambda i: (i, 0)),
        ],
        out_shape=[
            jax.ShapeDtypeStruct((n, D), jnp.float32),
            jax.ShapeDtypeStruct((n, D), jnp.float32),
        ],
    )(x, wa, wb)


def _mid_body(a0_ref, a1_ref, c0_ref, c1_ref, r_ref, b_ref, wa_ref, wb_ref,
              g_ref, rr_ref):
    cnt = c0_ref[:, 0] + c1_ref[:, 0]
    inv = 1.0 / jnp.maximum(cnt, 1.0)
    mean = (a0_ref[...] + a1_ref[...]) * inv[:, None]
    h = jnp.maximum(mean + r_ref[...] + b_ref[...][None, :], 0.0)
    dn = (((1,), (1,)), ((), ()))
    g_ref[...] = lax.dot_general(h, wa_ref[...], dn, preferred_element_type=jnp.float32)
    rr_ref[...] = lax.dot_general(h, wb_ref[...], dn, preferred_element_type=jnp.float32)


def _tc_mid(agg0, agg1, c0, c1, r, b, wa, wb, br=2048):
    n = r.shape[0]
    return pl.pallas_call(
        _mid_body,
        grid=(n // br,),
        in_specs=[
            pl.BlockSpec((br, D), lambda i: (i, 0)),
            pl.BlockSpec((br, D), lambda i: (i, 0)),
            pl.BlockSpec((br, D), lambda i: (i, 0)),
            pl.BlockSpec((br, D), lambda i: (i, 0)),
            pl.BlockSpec((br, D), lambda i: (i, 0)),
            pl.BlockSpec((D,), lambda i: (0,)),
            pl.BlockSpec((D, D), lambda i: (0, 0)),
            pl.BlockSpec((D, D), lambda i: (0, 0)),
        ],
        out_specs=[
            pl.BlockSpec((br, D), lambda i: (i, 0)),
            pl.BlockSpec((br, D), lambda i: (i, 0)),
        ],
        out_shape=[
            jax.ShapeDtypeStruct((n, D), jnp.float32),
            jax.ShapeDtypeStruct((n, D), jnp.float32),
        ],
    )(agg0, agg1, c0, c1, r, b, wa, wb)


def _final_body(a0_ref, a1_ref, c0_ref, c1_ref, r_ref, b_ref, o_ref):
    cnt = c0_ref[:, 0] + c1_ref[:, 0]
    inv = 1.0 / jnp.maximum(cnt, 1.0)
    mean = (a0_ref[...] + a1_ref[...]) * inv[:, None]
    o_ref[...] = mean + r_ref[...] + b_ref[...][None, :]


def _tc_final(agg0, agg1, c0, c1, r, b, br=2048):
    n = r.shape[0]
    return pl.pallas_call(
        _final_body,
        grid=(n // br,),
        in_specs=[
            pl.BlockSpec((br, D), lambda i: (i, 0)),
            pl.BlockSpec((br, D), lambda i: (i, 0)),
            pl.BlockSpec((br, D), lambda i: (i, 0)),
            pl.BlockSpec((br, D), lambda i: (i, 0)),
            pl.BlockSpec((br, D), lambda i: (i, 0)),
            pl.BlockSpec((D,), lambda i: (0,)),
        ],
        out_specs=pl.BlockSpec((br, D), lambda i: (i, 0)),
        out_shape=jax.ShapeDtypeStruct((n, D), jnp.float32),
    )(agg0, agg1, c0, c1, r, b)


CD = 128
NQ = 2


def _sc_body(n_chunks, g_hbm, src_hbm, dst_hbm, agg_out, *refs):
    sidx = list(refs[0:NQ])
    didx = list(refs[NQ:2 * NQ])
    rows = list(refs[2 * NQ:2 * NQ + 2])
    z_v = refs[2 * NQ + 2]
    acc_sh = refs[2 * NQ + 3]
    gsem = list(refs[2 * NQ + 4:2 * NQ + 6])
    isem = list(refs[2 * NQ + 6:2 * NQ + 6 + NQ])

    cid = lax.axis_index("c")
    sid = lax.axis_index("s")
    wid = cid * NS + sid
    r0 = sid * ROWS_PER_TILE
    base = wid * n_chunks

    def zfill(k, _):
        i = k // (D // 16)
        j = k % (D // 16)
        z_v[i, pl.ds(j * 16, 16)] = jnp.zeros((16,), jnp.float32)
        return 0
    lax.fori_loop(0, 16 * (D // 16), zfill, 0)

    def zinit(i, _):
        pltpu.sync_copy(z_v, acc_sh.at[pl.ds(r0 + i * 16, 16)])
        return 0
    lax.fori_loop(0, ROWS_PER_TILE // 16, zinit, 0)

    plsc.subcore_barrier()

    def issue_i(i, q):
        pltpu.async_copy(src_hbm.at[base + i], sidx[q], isem[q])
        pltpu.async_copy(dst_hbm.at[base + i], didx[q], isem[q])

    def wait_i(q):
        pltpu.make_async_copy(src_hbm.at[0], sidx[q], isem[q]).wait()
        pltpu.make_async_copy(dst_hbm.at[0], didx[q], isem[q]).wait()

    issue_i(0, 0)
    wait_i(0)
    pltpu.async_copy(g_hbm.at[sidx[0]], rows[0], gsem[0])
    issue_i(1, 1)

    def step(k, _):
        i0 = 2 * k
        wait_i(1)
        pltpu.async_copy(g_hbm.at[sidx[1]], rows[1], gsem[1])
        pltpu.make_async_copy(g_hbm.at[sidx[0]], rows[0], gsem[0]).wait()
        pltpu.sync_copy(rows[0], acc_sh.at[didx[0]], add=True)
        issue_i(i0 + 2, 0)
        pltpu.make_async_copy(g_hbm.at[sidx[1]], rows[1], gsem[1]).wait()
        pltpu.sync_copy(rows[1], acc_sh.at[didx[1]], add=True)
        issue_i(i0 + 3, 1)
        wait_i(0)
        pltpu.async_copy(g_hbm.at[sidx[0]], rows[0], gsem[0])
        return 0
    lax.fori_loop(0, n_chunks // 2, step, 0)

    pltpu.make_async_copy(g_hbm.at[sidx[0]], rows[0], gsem[0]).wait()
    wait_i(1)

    plsc.subcore_barrier()

    pltpu.sync_copy(acc_sh.at[pl.ds(r0, ROWS_PER_TILE)],
                    agg_out.at[pl.ds(cid * NP + r0, ROWS_PER_TILE)])


def _make_sc_segsum(n_chunks):
    mesh = plsc.VectorSubcoreMesh(core_axis_name="c", subcore_axis_name="s")
    out_type = jax.ShapeDtypeStruct((NC * NP, D), jnp.float32)
    scratch = (
        [pltpu.VMEM((CD,), jnp.int32) for _ in range(NQ)]
        + [pltpu.VMEM((CD,), jnp.int32) for _ in range(NQ)]
        + [pltpu.VMEM((CD, D), jnp.float32) for _ in range(2)]
        + [pltpu.VMEM((16, D), jnp.float32)]
        + [pltpu.VMEM_SHARED((NP, D), jnp.float32)]
        + [pltpu.SemaphoreType.DMA] * (2 + NQ)
    )
    return pl.kernel(functools.partial(_sc_body, n_chunks),
                     out_type=out_type, mesh=mesh, scratch_types=scratch)


def _sc_count_body(n_chunks, dst_hbm, cnt_out, dst_v, ones_v, z_v, acc_sh, sem):
    cid = lax.axis_index("c")
    sid = lax.axis_index("s")
    wid = cid * NS + sid
    r0 = sid * ROWS_PER_TILE

    def zfill(k, _):
        i = k // (D // 16)
        j = k % (D // 16)
        z_v[i, pl.ds(j * 16, 16)] = jnp.zeros((16,), jnp.float32)
        return 0
    lax.fori_loop(0, 16 * (D // 16), zfill, 0)

    def zinit(i, _):
        pltpu.sync_copy(z_v, acc_sh.at[pl.ds(r0 + i * 16, 16)])
        return 0
    lax.fori_loop(0, ROWS_PER_TILE // 16, zinit, 0)

    def onesfill(k, _):
        i = k // (D // 16)
        j = k % (D // 16)
        ones_v[i, pl.ds(j * 16, 16)] = jnp.ones((16,), jnp.float32)
        return 0
    lax.fori_loop(0, C2 * (D // 16), onesfill, 0)

    pltpu.sync_copy(dst_hbm.at[pl.ds(wid * n_chunks, n_chunks)], dst_v)

    plsc.subcore_barrier()

    def step(i, _):
        pltpu.sync_copy(ones_v, acc_sh.at[dst_v.at[i]], add=True)
        return 0
    lax.fori_loop(0, n_chunks, step, 0)

    plsc.subcore_barrier()

    pltpu.sync_copy(acc_sh.at[pl.ds(r0, ROWS_PER_TILE)],
                    cnt_out.at[pl.ds(cid * NP + r0, ROWS_PER_TILE)])


def _make_sc_count(n_chunks):
    mesh = plsc.VectorSubcoreMesh(core_axis_name="c", subcore_axis_name="s")
    out_type = jax.ShapeDtypeStruct((NC * NP, D), jnp.float32)
    scratch = [
        pltpu.VMEM((n_chunks, C2), jnp.int32),
        pltpu.VMEM((C2, D), jnp.float32),
        pltpu.VMEM((16, D), jnp.float32),
        pltpu.VMEM_SHARED((NP, D), jnp.float32),
        pltpu.SemaphoreType.DMA,
    ]
    return pl.kernel(functools.partial(_sc_count_body, n_chunks),
                     out_type=out_type, mesh=mesh, scratch_types=scratch)


def kernel(x, edge_index, W1l, b1, W1r, W2l, b2, W2r):
    E = edge_index.shape[1]
    n_chunks = -(-(-(-E // (NW * CD))) // NQ) * NQ
    e_pad = n_chunks * CD * NW
    src = jnp.concatenate([edge_index[0], jnp.zeros((e_pad - E,), jnp.int32)])
    dst = jnp.concatenate([edge_index[1], jnp.full((e_pad - E,), N, jnp.int32)])
    zrows = jnp.zeros((8, CD), jnp.int32)
    src2 = jnp.concatenate([src.reshape(NW * n_chunks, CD), zrows])
    dst2 = jnp.concatenate([dst.reshape(NW * n_chunks, CD), zrows])
    nc = -(-(-(-E // (NW * C2))) // 8) * 8
    ec_pad = nc * C2 * NW
    dstc = jnp.concatenate(
        [edge_index[1], jnp.full((ec_pad - E,), N, jnp.int32)]
    ).reshape(NW * nc, C2)
    x_p = jnp.pad(x, ((0, NP - N), (0, 0)))

    sc_segsum = _make_sc_segsum(n_chunks)
    sc_count = _make_sc_count(nc)

    cntp = sc_count(dstc)
    g1, r1 = _tc_dual_matmul(x_p, W1l, W1r)
    agg1p = sc_segsum(g1, src2, dst2)
    g2, r2 = _tc_mid(agg1p[:NP], agg1p[NP:], cntp[:NP], cntp[NP:],
                     r1, b1, W2l, W2r)
    agg2p = sc_segsum(g2, src2, dst2)
    out = _tc_final(agg2p[:NP], agg2p[NP:], cntp[:NP], cntp[NP:], r2, b2)
    return out[:N]


# speedup vs baseline: 1.5335x; 1.5335x over previous
import functools

import jax
import jax.numpy as jnp
from jax import lax
from jax.experimental import pallas as pl
from jax.experimental.pallas import tpu as pltpu
from jax.experimental.pallas import tpu_sc as plsc

N = 10000
D = 128
NP = 10240
NC = 2
NS = 16
NW = NC * NS
C = 64
C2 = 128
ROWS_PER_TILE = NP // NS


def _dual_matmul_body(x_ref, wa_ref, wb_ref, a_ref, b_ref):
    xv = x_ref[...]
    dn = (((1,), (1,)), ((), ()))
    a_ref[...] = lax.dot_general(xv, wa_ref[...], dn, preferred_element_type=jnp.float32)
    b_ref[...] = lax.dot_general(xv, wb_ref[...], dn, preferred_element_type=jnp.float32)


def _tc_dual_matmul(x, wa, wb, br=2048):
    n = x.shape[0]
    return pl.pallas_call(
        _dual_matmul_body,
        grid=(n // br,),
        in_specs=[
            pl.BlockSpec((br, D), lambda i: (i, 0)),
            pl.BlockSpec((D, D), lambda i: (0, 0)),
            pl.BlockSpec((D, D), lambda i: (0, 0)),
        ],
        out_specs=[
            pl.BlockSpec((br, D), lambda i: (i, 0)),
            pl.BlockSpec((br, D), lambda i: (i, 0)),
        ],
        out_shape=[
            jax.ShapeDtypeStruct((n, D), jnp.float32),
            jax.ShapeDtypeStruct((n, D), jnp.float32),
        ],
    )(x, wa, wb)


def _mid_body(a0_ref, a1_ref, c0_ref, c1_ref, r_ref, b_ref, wa_ref, wb_ref,
              g_ref, rr_ref):
    cnt = c0_ref[:, 0] + c1_ref[:, 0]
    inv = 1.0 / jnp.maximum(cnt, 1.0)
    mean = (a0_ref[...] + a1_ref[...]) * inv[:, None]
    h = jnp.maximum(mean + r_ref[...] + b_ref[...][None, :], 0.0)
    dn = (((1,), (1,)), ((), ()))
    g_ref[...] = lax.dot_general(h, wa_ref[...], dn, preferred_element_type=jnp.float32)
    rr_ref[...] = lax.dot_general(h, wb_ref[...], dn, preferred_element_type=jnp.float32)


def _tc_mid(agg0, agg1, c0, c1, r, b, wa, wb, br=2048):
    n = r.shape[0]
    return pl.pallas_call(
        _mid_body,
        grid=(n // br,),
        in_specs=[
            pl.BlockSpec((br, D), lambda i: (i, 0)),
            pl.BlockSpec((br, D), lambda i: (i, 0)),
            pl.BlockSpec((br, D), lambda i: (i, 0)),
            pl.BlockSpec((br, D), lambda i: (i, 0)),
            pl.BlockSpec((br, D), lambda i: (i, 0)),
            pl.BlockSpec((D,), lambda i: (0,)),
            pl.BlockSpec((D, D), lambda i: (0, 0)),
            pl.BlockSpec((D, D), lambda i: (0, 0)),
        ],
        out_specs=[
            pl.BlockSpec((br, D), lambda i: (i, 0)),
            pl.BlockSpec((br, D), lambda i: (i, 0)),
        ],
        out_shape=[
            jax.ShapeDtypeStruct((n, D), jnp.float32),
            jax.ShapeDtypeStruct((n, D), jnp.float32),
        ],
    )(agg0, agg1, c0, c1, r, b, wa, wb)


def _final_body(a0_ref, a1_ref, c0_ref, c1_ref, r_ref, b_ref, o_ref):
    cnt = c0_ref[:, 0] + c1_ref[:, 0]
    inv = 1.0 / jnp.maximum(cnt, 1.0)
    mean = (a0_ref[...] + a1_ref[...]) * inv[:, None]
    o_ref[...] = mean + r_ref[...] + b_ref[...][None, :]


def _tc_final(agg0, agg1, c0, c1, r, b, br=2048):
    n = r.shape[0]
    return pl.pallas_call(
        _final_body,
        grid=(n // br,),
        in_specs=[
            pl.BlockSpec((br, D), lambda i: (i, 0)),
            pl.BlockSpec((br, D), lambda i: (i, 0)),
            pl.BlockSpec((br, D), lambda i: (i, 0)),
            pl.BlockSpec((br, D), lambda i: (i, 0)),
            pl.BlockSpec((br, D), lambda i: (i, 0)),
            pl.BlockSpec((D,), lambda i: (0,)),
        ],
        out_specs=pl.BlockSpec((br, D), lambda i: (i, 0)),
        out_shape=jax.ShapeDtypeStruct((n, D), jnp.float32),
    )(agg0, agg1, c0, c1, r, b)


def _sc_body(n_chunks, g_hbm, src_hbm, dst_hbm, agg_out,
             sa_v, da_v, sb_v, db_v, rows_a, rows_b, z_v, acc_sh,
             sem_a, sem_b, sem_ia, sem_ib):
    cid = lax.axis_index("c")
    sid = lax.axis_index("s")
    wid = cid * NS + sid
    r0 = sid * ROWS_PER_TILE
    base = wid * n_chunks

    def zfill(k, _):
        i = k // (D // 16)
        j = k % (D // 16)
        z_v[i, pl.ds(j * 16, 16)] = jnp.zeros((16,), jnp.float32)
        return 0
    lax.fori_loop(0, 16 * (D // 16), zfill, 0)

    def zinit(i, _):
        pltpu.sync_copy(z_v, acc_sh.at[pl.ds(r0 + i * 16, 16)])
        return 0
    lax.fori_loop(0, ROWS_PER_TILE // 16, zinit, 0)

    plsc.subcore_barrier()

    def issue_i(i, s_v, d_v, sem):
        pltpu.async_copy(src_hbm.at[base + i], s_v, sem)
        pltpu.async_copy(dst_hbm.at[base + i], d_v, sem)

    def wait_i(s_v, d_v, sem):
        pltpu.make_async_copy(src_hbm.at[0], s_v, sem).wait()
        pltpu.make_async_copy(dst_hbm.at[0], d_v, sem).wait()

    def issue_g(s_v, rows, sem):
        pltpu.async_copy(g_hbm.at[s_v], rows, sem)

    def wait_g(s_v, rows, sem):
        pltpu.make_async_copy(g_hbm.at[s_v], rows, sem).wait()

    def scat(d_v, rows):
        pltpu.sync_copy(rows, acc_sh.at[d_v], add=True)

    issue_i(0, sa_v, da_v, sem_ia)
    wait_i(sa_v, da_v, sem_ia)
    issue_g(sa_v, rows_a, sem_a)
    issue_i(1, sb_v, db_v, sem_ib)

    pairs = (n_chunks - 1) // 2

    def step(k, _):
        i0 = 2 * k
        wait_i(sb_v, db_v, sem_ib)
        issue_g(sb_v, rows_b, sem_b)
        wait_g(sa_v, rows_a, sem_a)
        scat(da_v, rows_a)
        issue_i(i0 + 2, sa_v, da_v, sem_ia)
        wait_i(sa_v, da_v, sem_ia)
        issue_g(sa_v, rows_a, sem_a)
        wait_g(sb_v, rows_b, sem_b)
        scat(db_v, rows_b)
        issue_i(i0 + 3, sb_v, db_v, sem_ib)
        return 0
    lax.fori_loop(0, pairs, step, 0)

    wait_g(sa_v, rows_a, sem_a)
    scat(da_v, rows_a)
    wait_i(sb_v, db_v, sem_ib)

    plsc.subcore_barrier()

    pltpu.sync_copy(acc_sh.at[pl.ds(r0, ROWS_PER_TILE)],
                    agg_out.at[pl.ds(cid * NP + r0, ROWS_PER_TILE)])


def _make_sc_segsum(n_chunks):
    mesh = plsc.VectorSubcoreMesh(core_axis_name="c", subcore_axis_name="s")
    out_type = jax.ShapeDtypeStruct((NC * NP, D), jnp.float32)
    scratch = [
        pltpu.VMEM((C2,), jnp.int32),
        pltpu.VMEM((C2,), jnp.int32),
        pltpu.VMEM((C2,), jnp.int32),
        pltpu.VMEM((C2,), jnp.int32),
        pltpu.VMEM((C2, D), jnp.float32),
        pltpu.VMEM((C2, D), jnp.float32),
        pltpu.VMEM((16, D), jnp.float32),
        pltpu.VMEM_SHARED((NP, D), jnp.float32),
        pltpu.SemaphoreType.DMA,
        pltpu.SemaphoreType.DMA,
        pltpu.SemaphoreType.DMA,
        pltpu.SemaphoreType.DMA,
    ]
    return pl.kernel(functools.partial(_sc_body, n_chunks),
                     out_type=out_type, mesh=mesh, scratch_types=scratch)


def _sc_count_body(n_chunks, dst_hbm, cnt_out, dst_v, ones_v, z_v, acc_sh, sem):
    cid = lax.axis_index("c")
    sid = lax.axis_index("s")
    wid = cid * NS + sid
    r0 = sid * ROWS_PER_TILE

    def zfill(k, _):
        i = k // (D // 16)
        j = k % (D // 16)
        z_v[i, pl.ds(j * 16, 16)] = jnp.zeros((16,), jnp.float32)
        return 0
    lax.fori_loop(0, 16 * (D // 16), zfill, 0)

    def zinit(i, _):
        pltpu.sync_copy(z_v, acc_sh.at[pl.ds(r0 + i * 16, 16)])
        return 0
    lax.fori_loop(0, ROWS_PER_TILE // 16, zinit, 0)

    def onesfill(k, _):
        i = k // (D // 16)
        j = k % (D // 16)
        ones_v[i, pl.ds(j * 16, 16)] = jnp.ones((16,), jnp.float32)
        return 0
    lax.fori_loop(0, C2 * (D // 16), onesfill, 0)

    pltpu.sync_copy(dst_hbm.at[pl.ds(wid * n_chunks, n_chunks)], dst_v)

    plsc.subcore_barrier()

    def step(i, _):
        pltpu.sync_copy(ones_v, acc_sh.at[dst_v.at[i]], add=True)
        return 0
    lax.fori_loop(0, n_chunks, step, 0)

    plsc.subcore_barrier()

    pltpu.sync_copy(acc_sh.at[pl.ds(r0, ROWS_PER_TILE)],
                    cnt_out.at[pl.ds(cid * NP + r0, ROWS_PER_TILE)])


def _make_sc_count(n_chunks):
    mesh = plsc.VectorSubcoreMesh(core_axis_name="c", subcore_axis_name="s")
    out_type = jax.ShapeDtypeStruct((NC * NP, D), jnp.float32)
    scratch = [
        pltpu.VMEM((n_chunks, C2), jnp.int32),
        pltpu.VMEM((C2, D), jnp.float32),
        pltpu.VMEM((16, D), jnp.float32),
        pltpu.VMEM_SHARED((NP, D), jnp.float32),
        pltpu.SemaphoreType.DMA,
    ]
    return pl.kernel(functools.partial(_sc_count_body, n_chunks),
                     out_type=out_type, mesh=mesh, scratch_types=scratch)


def kernel(x, edge_index, W1l, b1, W1r, W2l, b2, W2r):
    E = edge_index.shape[1]
    n_chunks = -(-E // (NW * C2))
    if n_chunks % 2 == 0:
        n_chunks += 1
    e_pad = n_chunks * C2 * NW
    src = jnp.concatenate([edge_index[0], jnp.zeros((e_pad - E,), jnp.int32)])
    dst = jnp.concatenate([edge_index[1], jnp.full((e_pad - E,), N, jnp.int32)])
    zrows = jnp.zeros((8, C2), jnp.int32)
    src2 = jnp.concatenate([src.reshape(NW * n_chunks, C2), zrows])
    dst2 = jnp.concatenate([dst.reshape(NW * n_chunks, C2), zrows])
    nc = -(-(-(-E // (NW * C2))) // 8) * 8
    ec_pad = nc * C2 * NW
    dstc = jnp.concatenate(
        [edge_index[1], jnp.full((ec_pad - E,), N, jnp.int32)]
    ).reshape(NW * nc, C2)
    x_p = jnp.pad(x, ((0, NP - N), (0, 0)))

    sc_segsum = _make_sc_segsum(n_chunks)
    sc_count = _make_sc_count(nc)

    cntp = sc_count(dstc)
    g1, r1 = _tc_dual_matmul(x_p, W1l, W1r)
    agg1p = sc_segsum(g1, src2, dst2)
    g2, r2 = _tc_mid(agg1p[:NP], agg1p[NP:], cntp[:NP], cntp[NP:],
                     r1, b1, W2l, W2r)
    agg2p = sc_segsum(g2, src2, dst2)
    out = _tc_final(agg2p[:NP], agg2p[NP:], cntp[:NP], cntp[NP:], r2, b2)
    return out[:N]
